# baseline (speedup 1.0000x reference)
import functools

import jax
import jax.numpy as jnp
import numpy as np
from jax import lax
from jax.experimental import pallas as pl
from jax.experimental.pallas import tpu as pltpu
from jax.experimental.pallas import tpu_sc as plsc

NC = 2
NS = 16
NW = NC * NS
L = 16

CHUNK = 80
N_PAD = 10112
ROWS_PER_TILE = N_PAD // NS
STAGE_ROWS = 128
STAGE_CHUNKS = ((0, 128), (128, 128), (256, 128), (384, 128), (512, 120))


def _spmm_sc(src, dst, w, support):
  n, fw = support.shape
  f = 2 * fw
  e = src.shape[0]
  per_w = e // NW
  n_chunks = per_w // CHUNK
  assert per_w % CHUNK == 0 and n == N_PAD and f % L == 0

  mesh = plsc.VectorSubcoreMesh(core_axis_name="c", subcore_axis_name="s")

  @functools.partial(
      pl.kernel,
      out_type=jax.ShapeDtypeStruct((NC, N_PAD, f), jnp.float32),
      mesh=mesh,
      scratch_types=[
          pltpu.VMEM((n_chunks, CHUNK), jnp.int32),
          pltpu.VMEM((n_chunks, CHUNK), jnp.int32),
          pltpu.VMEM((n_chunks, CHUNK), jnp.float32),
          pltpu.VMEM((4, CHUNK, fw), jnp.int32),
          pltpu.VMEM((4, CHUNK, f), jnp.float32),
          pltpu.VMEM_SHARED((N_PAD, f), jnp.float32),
          pltpu.VMEM((STAGE_ROWS, f), jnp.float32),
          pltpu.SemaphoreType.DMA((4,)),
          pltpu.SemaphoreType.DMA((4,)),
      ],
      compiler_params=pltpu.CompilerParams(use_tc_tiling_on_sc=False,
                                           needs_layout_passes=False),
  )
  def spmm(src_hbm, dst_hbm, w_hbm, sup_hbm, out_hbm,
           src_i, dst_i, w_i, rows3, srows3, acc_sh, stage_v,
           sem_g, sem_s):
    cid = lax.axis_index("c")
    sid = lax.axis_index("s")
    wid = sid * NC + cid

    pltpu.sync_copy(src_hbm.at[wid], src_i)
    pltpu.sync_copy(dst_hbm.at[wid], dst_i)
    pltpu.sync_copy(w_hbm.at[wid], w_i)

    def zrow(i, _):
      for j in range(f // L):
        stage_v[i, pl.ds(j * L, L)] = jnp.zeros((L,), jnp.float32)
      return 0
    lax.fori_loop(0, STAGE_ROWS, zrow, 0)
    rbase = sid * ROWS_PER_TILE
    for off, sz in STAGE_CHUNKS:
      pltpu.sync_copy(stage_v.at[pl.ds(0, sz)],
                      acc_sh.at[pl.ds(rbase + off, sz)])
    plsc.subcore_barrier()

    def gather_start(k, par):
      pltpu.async_copy(sup_hbm.at[src_i.at[k]], rows3.at[par], sem_g.at[par])

    def gather_wait(k, par):
      pltpu.make_async_copy(sup_hbm.at[src_i.at[k]], rows3.at[par],
                            sem_g.at[par]).wait()

    def scat_start(k, par):
      pltpu.async_copy(srows3.at[par], acc_sh.at[dst_i.at[k]], sem_s.at[par],
                       add=True)

    def scat_wait(k, par):
      pltpu.make_async_copy(srows3.at[par], acc_sh.at[dst_i.at[k]],
                            sem_s.at[par]).wait()

    gather_start(0, 0)
    gather_start(1, 1)
    gather_start(2, 2)

    def chunk_body(k, _):
      par = lax.rem(k, 4)
      gather_wait(k, par)
      @pl.when(k + 3 < n_chunks)
      def _():
        gather_start(k + 3, lax.rem(k + 3, 4))
      @pl.when(k >= 4)
      def _():
        scat_wait(k - 4, par)
      def do_scale(ps):
        rv = rows3.at[ps]
        sv = srows3.at[ps]

        @plsc.parallel_loop(0, CHUNK // L, step=1, unroll=5)
        def grp(g):
          wvec = w_i[k, pl.ds(g * L, L)]
          for t in range(L):
            i = g * L + t
            wv = wvec[t]
            for j in range(f // (2 * L)):
              u = plsc.bitcast(rv[i, pl.ds(j * L, L)], jnp.bfloat16)
              a, b = plsc.unpack(u, format=plsc.PackFormat.INTERLEAVED)
              sv[i, pl.ds(j * 2 * L, L)] = a * wv
              sv[i, pl.ds(j * 2 * L + L, L)] = b * wv

      for ps in range(4):
        @pl.when(par == ps)
        def _(ps=ps):
          do_scale(ps)

      scat_start(k, par)
      return 0

    lax.fori_loop(0, n_chunks, chunk_body, 0)
    for tail in range(4):
      kk = n_chunks - 4 + tail
      scat_wait(kk, kk % 4)
    plsc.subcore_barrier()

    for off, sz in STAGE_CHUNKS:
      ro = rbase + off
      pltpu.sync_copy(acc_sh.at[pl.ds(ro, sz)], stage_v.at[pl.ds(0, sz)])
      pltpu.sync_copy(stage_v.at[pl.ds(0, sz)], out_hbm.at[cid, pl.ds(ro, sz)])

  src3 = src.reshape(NW, n_chunks, CHUNK)
  dst3 = dst.reshape(NW, n_chunks, CHUNK)
  w3 = w.reshape(NW, n_chunks, CHUNK)
  return spmm(src3, dst3, w3, support)


N_P4 = N_PAD // 4
_PBLK = 632


def _bd4(w):
  return jnp.kron(jnp.eye(4, dtype=w.dtype), w)


def _pack_words(lo, hi):
  lo16 = jax.lax.bitcast_convert_type(lo.astype(jnp.bfloat16), jnp.uint16)
  hi16 = jax.lax.bitcast_convert_type(hi.astype(jnp.bfloat16), jnp.uint16)
  w = lo16.astype(jnp.uint32) | (hi16.astype(jnp.uint32) << 16)
  return jax.lax.bitcast_convert_type(w, jnp.int32)


def _mm_tc(x_pack, wa, wb):
  n, k4 = x_pack.shape
  m4 = wa.shape[1]

  def body(x_ref, wa_ref, wb_ref, o_ref):
    xb = x_ref[...]
    lo = jnp.dot(xb, wa_ref[...], preferred_element_type=jnp.float32)
    hi = jnp.dot(xb, wb_ref[...], preferred_element_type=jnp.float32)
    o_ref[...] = _pack_words(lo, hi)

  return pl.pallas_call(
      body,
      grid=(n // _PBLK,),
      in_specs=[
          pl.BlockSpec((_PBLK, k4), lambda i: (i, 0)),
          pl.BlockSpec((k4, m4), lambda i: (0, 0)),
          pl.BlockSpec((k4, m4), lambda i: (0, 0)),
      ],
      out_specs=pl.BlockSpec((_PBLK, m4), lambda i: (i, 0)),
      out_shape=jax.ShapeDtypeStruct((n, m4), jnp.int32),
  )(x_pack, wa, wb)


def _relu_mm_tc(p_pack, wa, wb):
  _, n, f4 = p_pack.shape
  m4 = wa.shape[1]

  def body(p_ref, wa_ref, wb_ref, x_ref, s_ref):
    xb = jnp.maximum(p_ref[0] + p_ref[1], 0.0)
    x_ref[...] = xb
    lo = jnp.dot(xb, wa_ref[...], preferred_element_type=jnp.float32)
    hi = jnp.dot(xb, wb_ref[...], preferred_element_type=jnp.float32)
    s_ref[...] = _pack_words(lo, hi)

  return pl.pallas_call(
      body,
      grid=(n // _PBLK,),
      in_specs=[
          pl.BlockSpec((2, _PBLK, f4), lambda i: (0, i, 0)),
          pl.BlockSpec((f4, m4), lambda i: (0, 0)),
          pl.BlockSpec((f4, m4), lambda i: (0, 0)),
      ],
      out_specs=[
          pl.BlockSpec((_PBLK, f4), lambda i: (i, 0)),
          pl.BlockSpec((_PBLK, m4), lambda i: (i, 0)),
      ],
      out_shape=[
          jax.ShapeDtypeStruct((n, f4), jnp.float32),
          jax.ShapeDtypeStruct((n, m4), jnp.int32),
      ],
  )(p_pack, wa, wb)


def _final_tc(p3, x1, x2, lws, b4):
  _, n, f4 = p3.shape
  c4 = lws.shape[2]
  ncls = c4 // 4

  def body(p_ref, x1_ref, x2_ref, w_ref, b_ref, o_ref):
    x3 = p_ref[0] + p_ref[1]
    h = (jnp.dot(x1_ref[...], w_ref[0], preferred_element_type=jnp.float32)
         + jnp.dot(x2_ref[...], w_ref[1], preferred_element_type=jnp.float32)
         + jnp.dot(x3, w_ref[2], preferred_element_type=jnp.float32)
         + b_ref[...])
    for q in range(4):
      hh = h[:, q * ncls:(q + 1) * ncls]
      m = jnp.max(hh, axis=1, keepdims=True)
      ex = jnp.exp(hh - m)
      o_ref[:, q * ncls:(q + 1) * ncls] = (
          hh - m - jnp.log(jnp.sum(ex, axis=1, keepdims=True)))

  return pl.pallas_call(
      body,
      grid=(n // _PBLK,),
      in_specs=[
          pl.BlockSpec((2, _PBLK, f4), lambda i: (0, i, 0)),
          pl.BlockSpec((_PBLK, f4), lambda i: (i, 0)),
          pl.BlockSpec((_PBLK, f4), lambda i: (i, 0)),
          pl.BlockSpec((3, f4, c4), lambda i: (0, 0, 0)),
          pl.BlockSpec((1, c4), lambda i: (0, 0)),
      ],
      out_specs=pl.BlockSpec((_PBLK, c4), lambda i: (i, 0)),
      out_shape=jax.ShapeDtypeStruct((n, c4), jnp.float32),
  )(p3, x1, x2, lws, b4)


_UNPACK_PERM = np.concatenate([
    b * 32 + np.where(np.arange(32) % 2 == 0,
                      np.arange(32) // 2,
                      16 + np.arange(32) // 2)
    for b in range(2)
])


def kernel(x, edge_index, edge_weight, W1, W2, W3, lin_W, lin_b):
  src = edge_index[0]
  dst = edge_index[1]
  n, nfeat = x.shape
  f = W1.shape[1]
  ncls = lin_W.shape[1]
  xp = jnp.pad(x, ((0, N_PAD - n), (0, 0)))
  x4 = xp.reshape(N_P4, 4 * nfeat)
  perm = jnp.asarray(_UNPACK_PERM)
  W1p, W2p, W3p = W1[:, perm], W2[:, perm], W3[:, perm]
  WA1, WB1 = _bd4(W1p[:, 0::2]), _bd4(W1p[:, 1::2])
  WA2, WB2 = _bd4(W2p[:, 0::2]), _bd4(W2p[:, 1::2])
  WA3, WB3 = _bd4(W3p[:, 0::2]), _bd4(W3p[:, 1::2])
  lws = jnp.stack([_bd4(lin_W[i * f:(i + 1) * f]) for i in range(3)])
  b4 = jnp.concatenate([lin_b] * 4).reshape(1, 4 * ncls)

  s1 = _mm_tc(x4, WA1, WB1)
  p1 = _spmm_sc(src, dst, edge_weight, s1.reshape(N_PAD, f // 2))
  x1, s2 = _relu_mm_tc(p1.reshape(2, N_P4, 4 * f), WA2, WB2)
  p2 = _spmm_sc(src, dst, edge_weight, s2.reshape(N_PAD, f // 2))
  x2, s3 = _relu_mm_tc(p2.reshape(2, N_P4, 4 * f), WA3, WB3)
  p3 = _spmm_sc(src, dst, edge_weight, s3.reshape(N_PAD, f // 2))
  out4 = _final_tc(p3.reshape(2, N_P4, 4 * f), x1, x2, lws, b4)
  return out4.reshape(N_PAD, ncls)[:n]


# speedup vs baseline: 1.0332x; 1.0332x over previous
import functools

import jax
import jax.numpy as jnp
import numpy as np
from jax import lax
from jax.experimental import pallas as pl
from jax.experimental.pallas import tpu as pltpu
from jax.experimental.pallas import tpu_sc as plsc

NC = 2
NS = 16
NW = NC * NS
L = 16

CHUNK = 80
N_PAD = 10112
ROWS_PER_TILE = N_PAD // NS
STAGE_ROWS = 128
STAGE_CHUNKS = ((0, 128), (128, 128), (256, 128), (384, 128), (512, 120))


def _spmm_sc(src, dst, w, support):
  n, f = support.shape
  e = src.shape[0]
  per_w = e // NW
  n_chunks = per_w // CHUNK
  assert per_w % CHUNK == 0 and n == N_PAD and f % L == 0

  mesh = plsc.VectorSubcoreMesh(core_axis_name="c", subcore_axis_name="s")

  @functools.partial(
      pl.kernel,
      out_type=jax.ShapeDtypeStruct((NC, N_PAD, f), jnp.float32),
      mesh=mesh,
      scratch_types=[
          pltpu.VMEM((n_chunks, CHUNK), jnp.int32),
          pltpu.VMEM((n_chunks, CHUNK), jnp.int32),
          pltpu.VMEM((n_chunks, CHUNK), jnp.float32),
          pltpu.VMEM((4, CHUNK, f), jnp.bfloat16),
          pltpu.VMEM((4, CHUNK, f), jnp.float32),
          pltpu.VMEM_SHARED((N_PAD, f), jnp.float32),
          pltpu.VMEM((STAGE_ROWS, f), jnp.float32),
          pltpu.SemaphoreType.DMA((4,)),
          pltpu.SemaphoreType.DMA((4,)),
      ],
      compiler_params=pltpu.CompilerParams(use_tc_tiling_on_sc=False,
                                           needs_layout_passes=False),
  )
  def spmm(src_hbm, dst_hbm, w_hbm, sup_hbm, out_hbm,
           src_i, dst_i, w_i, rows3, srows3, acc_sh, stage_v,
           sem_g, sem_s):
    cid = lax.axis_index("c")
    sid = lax.axis_index("s")
    wid = sid * NC + cid

    pltpu.sync_copy(src_hbm.at[wid], src_i)
    pltpu.sync_copy(dst_hbm.at[wid], dst_i)
    pltpu.sync_copy(w_hbm.at[wid], w_i)

    def zrow(i, _):
      for j in range(f // L):
        stage_v[i, pl.ds(j * L, L)] = jnp.zeros((L,), jnp.float32)
      return 0
    lax.fori_loop(0, STAGE_ROWS, zrow, 0)
    rbase = sid * ROWS_PER_TILE
    for off, sz in STAGE_CHUNKS:
      pltpu.sync_copy(stage_v.at[pl.ds(0, sz)],
                      acc_sh.at[pl.ds(rbase + off, sz)])
    plsc.subcore_barrier()

    def gather_start(k, par):
      pltpu.async_copy(sup_hbm.at[src_i.at[k]], rows3.at[par], sem_g.at[par])

    def gather_wait(k, par):
      pltpu.make_async_copy(sup_hbm.at[src_i.at[k]], rows3.at[par],
                            sem_g.at[par]).wait()

    def scat_start(k, par):
      pltpu.async_copy(srows3.at[par], acc_sh.at[dst_i.at[k]], sem_s.at[par],
                       add=True)

    def scat_wait(k, par):
      pltpu.make_async_copy(srows3.at[par], acc_sh.at[dst_i.at[k]],
                            sem_s.at[par]).wait()

    gather_start(0, 0)
    gather_start(1, 1)
    gather_start(2, 2)

    def chunk_body(k, _):
      par = lax.rem(k, 4)
      gather_wait(k, par)
      @pl.when(k + 3 < n_chunks)
      def _():
        gather_start(k + 3, lax.rem(k + 3, 4))
      @pl.when(k >= 4)
      def _():
        scat_wait(k - 4, par)
      def do_scale(ps):
        rv = rows3.at[ps]
        sv = srows3.at[ps]

        @plsc.parallel_loop(0, CHUNK // L, step=1, unroll=5)
        def grp(g):
          wvec = w_i[k, pl.ds(g * L, L)]
          for t in range(L):
            i = g * L + t
            wv = wvec[t]
            for j in range(f // (2 * L)):
              u = rv[i, pl.ds(j * 2 * L, 2 * L)]
              a, b = plsc.unpack(u, format=plsc.PackFormat.INTERLEAVED)
              sv[i, pl.ds(j * 2 * L, L)] = a * wv
              sv[i, pl.ds(j * 2 * L + L, L)] = b * wv

      for ps in range(4):
        @pl.when(par == ps)
        def _(ps=ps):
          do_scale(ps)

      scat_start(k, par)
      return 0

    lax.fori_loop(0, n_chunks, chunk_body, 0)
    for tail in range(4):
      kk = n_chunks - 4 + tail
      scat_wait(kk, kk % 4)
    plsc.subcore_barrier()

    for off, sz in STAGE_CHUNKS:
      ro = rbase + off
      pltpu.sync_copy(acc_sh.at[pl.ds(ro, sz)], stage_v.at[pl.ds(0, sz)])
      pltpu.sync_copy(stage_v.at[pl.ds(0, sz)], out_hbm.at[cid, pl.ds(ro, sz)])

  src3 = src.reshape(NW, n_chunks, CHUNK)
  dst3 = dst.reshape(NW, n_chunks, CHUNK)
  w3 = w.reshape(NW, n_chunks, CHUNK)
  return spmm(src3, dst3, w3, support)


N_PK = N_PAD // 2
_PBLK = 1264


def _blockdiag2(w):
  z = jnp.zeros_like(w)
  return jnp.concatenate(
      [jnp.concatenate([w, z], axis=1), jnp.concatenate([z, w], axis=1)],
      axis=0)


def _mm_tc(x_pack, w_stack):
  n, k2 = x_pack.shape
  _, m2 = w_stack.shape

  def body(x_ref, w_ref, o_ref):
    o_ref[...] = jnp.dot(x_ref[...], w_ref[...],
                         preferred_element_type=jnp.float32
                         ).astype(jnp.bfloat16)

  return pl.pallas_call(
      body,
      grid=(n // _PBLK,),
      in_specs=[
          pl.BlockSpec((_PBLK, k2), lambda i: (i, 0)),
          pl.BlockSpec((k2, m2), lambda i: (0, 0)),
      ],
      out_specs=pl.BlockSpec((_PBLK, m2), lambda i: (i, 0)),
      out_shape=jax.ShapeDtypeStruct((n, m2), jnp.bfloat16),
  )(x_pack, w_stack)


def _relu_mm_tc(p_pack, w_stack):
  _, n, f2 = p_pack.shape
  _, m2 = w_stack.shape

  def body(p_ref, w_ref, x_ref, s_ref):
    xb = jnp.maximum(p_ref[0] + p_ref[1], 0.0)
    x_ref[...] = xb
    s_ref[...] = jnp.dot(xb, w_ref[...], preferred_element_type=jnp.float32
                         ).astype(jnp.bfloat16)

  return pl.pallas_call(
      body,
      grid=(n // _PBLK,),
      in_specs=[
          pl.BlockSpec((2, _PBLK, f2), lambda i: (0, i, 0)),
          pl.BlockSpec((f2, m2), lambda i: (0, 0)),
      ],
      out_specs=[
          pl.BlockSpec((_PBLK, f2), lambda i: (i, 0)),
          pl.BlockSpec((_PBLK, m2), lambda i: (i, 0)),
      ],
      out_shape=[
          jax.ShapeDtypeStruct((n, f2), jnp.float32),
          jax.ShapeDtypeStruct((n, m2), jnp.bfloat16),
      ],
  )(p_pack, w_stack)


def _final_tc(p3, x1, x2, lws, b2):
  _, n, f2 = p3.shape
  c2 = lws.shape[2]
  ncls = c2 // 2

  def body(p_ref, x1_ref, x2_ref, w_ref, b_ref, o_ref):
    x3 = p_ref[0] + p_ref[1]
    h = (jnp.dot(x1_ref[...], w_ref[0], preferred_element_type=jnp.float32)
         + jnp.dot(x2_ref[...], w_ref[1], preferred_element_type=jnp.float32)
         + jnp.dot(x3, w_ref[2], preferred_element_type=jnp.float32)
         + b_ref[...])
    for half in range(2):
      hh = h[:, half * ncls:(half + 1) * ncls]
      m = jnp.max(hh, axis=1, keepdims=True)
      ex = jnp.exp(hh - m)
      o_ref[:, half * ncls:(half + 1) * ncls] = (
          hh - m - jnp.log(jnp.sum(ex, axis=1, keepdims=True)))

  return pl.pallas_call(
      body,
      grid=(n // _PBLK,),
      in_specs=[
          pl.BlockSpec((2, _PBLK, f2), lambda i: (0, i, 0)),
          pl.BlockSpec((_PBLK, f2), lambda i: (i, 0)),
          pl.BlockSpec((_PBLK, f2), lambda i: (i, 0)),
          pl.BlockSpec((3, f2, c2), lambda i: (0, 0, 0)),
          pl.BlockSpec((1, c2), lambda i: (0, 0)),
      ],
      out_specs=pl.BlockSpec((_PBLK, c2), lambda i: (i, 0)),
      out_shape=jax.ShapeDtypeStruct((n, c2), jnp.float32),
  )(p3, x1, x2, lws, b2)


_UNPACK_PERM = np.concatenate([
    b * 32 + np.where(np.arange(32) % 2 == 0,
                      np.arange(32) // 2,
                      16 + np.arange(32) // 2)
    for b in range(2)
])


def kernel(x, edge_index, edge_weight, W1, W2, W3, lin_W, lin_b):
  src = edge_index[0]
  dst = edge_index[1]
  n, nfeat = x.shape
  f = W1.shape[1]
  ncls = lin_W.shape[1]
  xp = jnp.pad(x, ((0, N_PAD - n), (0, 0)))
  x_pack = xp.reshape(N_PK, 2 * nfeat)
  perm = jnp.asarray(_UNPACK_PERM)
  W1s = _blockdiag2(W1[:, perm])
  W2s = _blockdiag2(W2[:, perm])
  W3s = _blockdiag2(W3[:, perm])
  lws = jnp.stack([_blockdiag2(lin_W[i * f:(i + 1) * f]) for i in range(3)])
  b2 = jnp.concatenate([lin_b, lin_b]).reshape(1, 2 * ncls)

  s1 = _mm_tc(x_pack, W1s)
  p1 = _spmm_sc(src, dst, edge_weight, s1.reshape(N_PAD, f))
  x1, s2 = _relu_mm_tc(p1.reshape(2, N_PK, 2 * f), W2s)
  p2 = _spmm_sc(src, dst, edge_weight, s2.reshape(N_PAD, f))
  x2, s3 = _relu_mm_tc(p2.reshape(2, N_PK, 2 * f), W3s)
  p3 = _spmm_sc(src, dst, edge_weight, s3.reshape(N_PAD, f))
  out_pack = _final_tc(p3.reshape(2, N_PK, 2 * f), x1, x2, lws, b2)
  return out_pack.reshape(N_PAD, ncls)[:n]
